# baseline (speedup 1.0000x reference)
import functools

import jax
import jax.numpy as jnp
import numpy as np
from jax.experimental import pallas as pl
from jax.experimental.pallas import tpu as pltpu
from jax.experimental.pallas import tpu_sc as plsc

_E = 8
_D = 1024
_F = 2048
_T = 128
_GROUP = 128
TOPK_ = 2
_FB = 1024

_P_ACT = (np.arange(_FB)[:, None]
          == (((np.arange(_FB)[None, :] % (_FB // 4)) // 2) * 8
              + 2 * (np.arange(_FB)[None, :] // (_FB // 4))
              + np.arange(_FB)[None, :] % 2)).astype(jnp.bfloat16)


def _route_kernel(ids0_hbm, ids1_hbm, tw0_hbm, tw1_hbm, out_hbm,
                  i0_v, i1_v, w0_v, w1_v, out_v):
  wid = jax.lax.axis_index("s") * 2 + jax.lax.axis_index("c")
  e_id = wid >> 2
  tbase = (wid & 3) * 32
  pltpu.sync_copy(ids0_hbm, i0_v)
  pltpu.sync_copy(ids1_hbm, i1_v)
  pltpu.sync_copy(tw0_hbm, w0_v)
  pltpu.sync_copy(tw1_hbm, w1_v)
  for v in range(2):
    sl = pl.ds(tbase + v * 16, 16)
    r = (jnp.where(i0_v[sl] == e_id, w0_v[sl], 0.0)
         + jnp.where(i1_v[sl] == e_id, w1_v[sl], 0.0))
    out_v[pl.ds(v * 16, 16)] = r
  pltpu.sync_copy(out_v, out_hbm.at[pl.ds(wid * 32, 32)])


def _moe_kernel(wemat_ref, x_ref, p_ref, wg_ref, sg_ref,
                wu_ref, su_ref, w2_ref, s2_ref, out_ref):
  e = pl.program_id(0)
  j = pl.program_id(1)

  shifts = (jnp.arange(8, dtype=jnp.int32) * 4)[None, :, None]

  def srep_rows(s, k8, groups):
    n = s.shape[-1]
    srep = jnp.broadcast_to(s[:, None, :], (groups, _GROUP // 8, n))
    return srep.reshape(k8, n)

  def dequant(q, s, k8, groups):
    n = q.shape[-1]
    nib = ((q[:, None, :] >> shifts) & 0xF).astype(jnp.float32)
    w = (nib - 8.0) * srep_rows(s, k8, groups)[:, None, :]
    return w.reshape(k8 * 8, n).astype(jnp.bfloat16)

  def dequant_planes(q, s, k8, groups):
    n = q.shape[-1]
    sbits = ((jax.lax.bitcast_convert_type(s, jnp.int32) + 0x8000) >> 16)
    sword = srep_rows((sbits << 16) | sbits, k8, groups)
    spk = pltpu.bitcast(sword, jnp.bfloat16)
    planes = []
    for j in range(4):
      lo = (q >> (8 * j)) & 0xF
      if j < 2:
        hi = (q << (12 - 8 * j)) & 0xF0000
      else:
        hi = (q >> (8 * j - 12)) & 0xF0000
      w = lo | hi | 0x43004300
      wb = pltpu.bitcast(w, jnp.bfloat16)
      planes.append((wb - jnp.bfloat16(136.0)) * spk)
    return jnp.concatenate(planes, axis=0)

  x = x_ref[...]
  wg = dequant_planes(wg_ref[0], sg_ref[0], _D // 8, _D // _GROUP)
  hg = jnp.dot(x, wg, preferred_element_type=jnp.float32)
  wu = dequant_planes(wu_ref[0], su_ref[0], _D // 8, _D // _GROUP)
  hu = jnp.dot(x, wu, preferred_element_type=jnp.float32)

  lane_e = jax.lax.broadcasted_iota(jnp.int32, (_T, _E), 1)
  we = jnp.sum(jnp.where(lane_e == e, wemat_ref[...], 0.0), axis=1,
               keepdims=True)
  act = (hg * jax.nn.sigmoid(hg) * hu * we).astype(jnp.bfloat16)
  act = jnp.dot(act, p_ref[...], preferred_element_type=jnp.float32)
  act = act.astype(jnp.bfloat16)

  w2 = dequant_planes(w2_ref[0], s2_ref[0, 0], _FB // 8, _FB // _GROUP)
  o = jnp.dot(act, w2, preferred_element_type=jnp.float32)

  @pl.when((e == 0) & (j == 0))
  def _init():
    out_ref[...] = jnp.zeros_like(out_ref)

  out_ref[...] += o


@jax.jit
def kernel(x, topk_ids, topk_weights, w13_qweight, w13_scales, w2_qweight,
           w2_scales):
  jblocks = _F // _FB
  grid = (_E, jblocks)

  route = functools.partial(
      pl.kernel,
      mesh=plsc.VectorSubcoreMesh(core_axis_name="c", subcore_axis_name="s"),
      out_type=jax.ShapeDtypeStruct((_E * _T,), jnp.float32),
      scratch_types=[
          pltpu.VMEM((_T,), jnp.int32),
          pltpu.VMEM((_T,), jnp.int32),
          pltpu.VMEM((_T,), jnp.float32),
          pltpu.VMEM((_T,), jnp.float32),
          pltpu.VMEM((32,), jnp.float32),
      ],
  )(_route_kernel)
  ids32 = topk_ids.astype(jnp.int32)
  we_mat = route(
      ids32[:, 0], ids32[:, 1], topk_weights[:, 0], topk_weights[:, 1],
  ).reshape(_E, _T).T

  out = pl.pallas_call(
      _moe_kernel,
      grid=grid,
      in_specs=[
          pl.BlockSpec((_T, _E), lambda e, j: (0, 0)),
          pl.BlockSpec((_T, _D), lambda e, j: (0, 0)),
          pl.BlockSpec((_FB, _FB), lambda e, j: (0, 0)),
          pl.BlockSpec((1, _D // 8, _FB), lambda e, j: (e, 0, j)),
          pl.BlockSpec((1, _D // _GROUP, _FB), lambda e, j: (e, 0, j)),
          pl.BlockSpec((1, _D // 8, _FB), lambda e, j: (e, 0, j + jblocks)),
          pl.BlockSpec((1, _D // _GROUP, _FB), lambda e, j: (e, 0, j + jblocks)),
          pl.BlockSpec((1, _FB // 8, _D), lambda e, j: (e, j, 0)),
          pl.BlockSpec((1, 1, _FB // _GROUP, _D), lambda e, j: (e, j, 0, 0)),
      ],
      out_specs=pl.BlockSpec((_T, _D), lambda e, j: (0, 0)),
      out_shape=jax.ShapeDtypeStruct((_T, _D), jnp.float32),
      compiler_params=pltpu.CompilerParams(
          dimension_semantics=("arbitrary", "arbitrary"),
      ),
  )(
      we_mat,
      x.reshape(_T, _D // 8, 4, 2).transpose(0, 2, 1, 3).reshape(_T, _D)
      .astype(jnp.bfloat16),
      _P_ACT,
      w13_qweight,
      w13_scales,
      w13_qweight,
      w13_scales,
      w2_qweight,
      w2_scales.reshape(_E, jblocks, _FB // _GROUP, _D),
  )
  return out


# speedup vs baseline: 1.3566x; 1.3566x over previous
import jax
import jax.numpy as jnp
import numpy as np
from jax.experimental import pallas as pl
from jax.experimental.pallas import tpu as pltpu

_E = 8
_D = 1024
_F = 2048
_T = 128
_GROUP = 128
TOPK_ = 2
_FB = 1024

_P_ACT = (np.arange(_FB)[:, None]
          == (((np.arange(_FB)[None, :] % (_FB // 4)) // 2) * 8
              + 2 * (np.arange(_FB)[None, :] // (_FB // 4))
              + np.arange(_FB)[None, :] % 2)).astype(jnp.bfloat16)


def _moe_kernel(ids_ref, tw_ref, x_ref, p_ref, wg_ref, sg_ref,
                wu_ref, su_ref, w2_ref, s2_ref, out_ref):
  e = pl.program_id(0)
  j = pl.program_id(1)

  shifts = (jnp.arange(8, dtype=jnp.int32) * 4)[None, :, None]

  def srep_rows(s, k8, groups):
    n = s.shape[-1]
    srep = jnp.broadcast_to(s[:, None, :], (groups, _GROUP // 8, n))
    return srep.reshape(k8, n)

  def dequant(q, s, k8, groups):
    n = q.shape[-1]
    nib = ((q[:, None, :] >> shifts) & 0xF).astype(jnp.float32)
    w = (nib - 8.0) * srep_rows(s, k8, groups)[:, None, :]
    return w.reshape(k8 * 8, n).astype(jnp.bfloat16)

  def dequant_planes(q, s, k8, groups):
    n = q.shape[-1]
    sbits = ((jax.lax.bitcast_convert_type(s, jnp.int32) + 0x8000) >> 16)
    sword = srep_rows((sbits << 16) | sbits, k8, groups)
    spk = pltpu.bitcast(sword, jnp.bfloat16)
    planes = []
    for j in range(4):
      lo = (q >> (8 * j)) & 0xF
      if j < 2:
        hi = (q << (12 - 8 * j)) & 0xF0000
      else:
        hi = (q >> (8 * j - 12)) & 0xF0000
      w = lo | hi | 0x43004300
      wb = pltpu.bitcast(w, jnp.bfloat16)
      planes.append((wb - jnp.bfloat16(136.0)) * spk)
    return jnp.concatenate(planes, axis=0)

  x = x_ref[...]
  wg = dequant_planes(wg_ref[0], sg_ref[0], _D // 8, _D // _GROUP)
  hg = jnp.dot(x, wg, preferred_element_type=jnp.float32)
  wu = dequant_planes(wu_ref[0], su_ref[0], _D // 8, _D // _GROUP)
  hu = jnp.dot(x, wu, preferred_element_type=jnp.float32)

  we = jnp.sum(jnp.where(ids_ref[...] == e, tw_ref[...], 0.0), axis=1,
               keepdims=True)
  act = (hg * jax.nn.sigmoid(hg) * hu * we).astype(jnp.bfloat16)
  act = jnp.dot(act, p_ref[...], preferred_element_type=jnp.float32)
  act = act.astype(jnp.bfloat16)

  w2 = dequant_planes(w2_ref[0], s2_ref[0, 0], _FB // 8, _FB // _GROUP)
  o = jnp.dot(act, w2, preferred_element_type=jnp.float32)

  @pl.when((e == 0) & (j == 0))
  def _init():
    out_ref[...] = jnp.zeros_like(out_ref)

  out_ref[...] += o


@jax.jit
def kernel(x, topk_ids, topk_weights, w13_qweight, w13_scales, w2_qweight,
           w2_scales):
  jblocks = _F // _FB
  grid = (_E, jblocks)

  out = pl.pallas_call(
      _moe_kernel,
      grid=grid,
      in_specs=[
          pl.BlockSpec((_T, 2), lambda e, j: (0, 0)),
          pl.BlockSpec((_T, 2), lambda e, j: (0, 0)),
          pl.BlockSpec((_T, _D), lambda e, j: (0, 0)),
          pl.BlockSpec((_FB, _FB), lambda e, j: (0, 0)),
          pl.BlockSpec((1, _D // 8, _FB), lambda e, j: (e, 0, j)),
          pl.BlockSpec((1, _D // _GROUP, _FB), lambda e, j: (e, 0, j)),
          pl.BlockSpec((1, _D // 8, _FB), lambda e, j: (e, 0, j + jblocks)),
          pl.BlockSpec((1, _D // _GROUP, _FB), lambda e, j: (e, 0, j + jblocks)),
          pl.BlockSpec((1, _FB // 8, _D), lambda e, j: (e, j, 0)),
          pl.BlockSpec((1, 1, _FB // _GROUP, _D), lambda e, j: (e, j, 0, 0)),
      ],
      out_specs=pl.BlockSpec((_T, _D), lambda e, j: (0, 0)),
      out_shape=jax.ShapeDtypeStruct((_T, _D), jnp.float32),
      compiler_params=pltpu.CompilerParams(
          dimension_semantics=("arbitrary", "arbitrary"),
      ),
  )(
      topk_ids.astype(jnp.int32),
      topk_weights,
      x.reshape(_T, _D // 8, 4, 2).transpose(0, 2, 1, 3).reshape(_T, _D)
      .astype(jnp.bfloat16),
      _P_ACT,
      w13_qweight,
      w13_scales,
      w13_qweight,
      w13_scales,
      w2_qweight,
      w2_scales.reshape(_E, jblocks, _FB // _GROUP, _D),
  )
  return out
